# baseline (speedup 1.0000x reference)
import functools

import jax
import jax.numpy as jnp
from jax import lax
from jax.experimental import pallas as pl
from jax.experimental.pallas import tpu as pltpu
from jax.experimental.pallas import tpu_sc as plsc

BATCH = 1024
NUM_CLASSES = 100000

SC_ROWS = 0
_PROBE_XLA_GATHER = True
TC_ROWS = BATCH - SC_ROWS

NC = 2
NS = 16
L = 16
NW = NC * NS
G = 8
NBUF = 6
CC = 1280
NCH = 78
LASTC = 128
TAILW = 32
NPAIR = 8
UNIT = 8 * L
_INT_MAX = 0x7FFFFFFF

TC_BR = 8
TC_BC = 16384
TC_NJ = -(-NUM_CLASSES // TC_BC)
TC_NI = TC_ROWS // TC_BR
TC_NACC = 8
TC_NVIEW = 4
TC_VW = 25088


def _scan_rows(buf, maxacc, idxacc, colbase, cols):
  full = cols // UNIT
  rem = (cols - full * UNIT) // L
  step0 = colbase // UNIT
  for rr in range(G):
    accs = tuple(maxacc[rr, pl.ds(p * L, L)] for p in range(NPAIR))
    idxs = tuple(idxacc[rr, pl.ds(p * L, L)] for p in range(NPAIR))
    vstep = jnp.full((L,), step0, jnp.int32)

    def body(off, st, rr=rr):
      accs, idxs, vstep = st
      accs, idxs = list(accs), list(idxs)
      for k in range(NPAIR):
        v = buf[rr, pl.ds(off + k * L, L)]
        m = v > accs[k]
        accs[k] = jnp.maximum(accs[k], v)
        idxs[k] = jnp.where(m, vstep, idxs[k])
      return (tuple(accs), tuple(idxs), vstep + 1)

    if full > 0:
      accs, idxs, vstep = plsc.parallel_loop(
          0, full * UNIT, step=UNIT, carry=(accs, idxs, vstep))(body)
    accs, idxs = list(accs), list(idxs)
    for k in range(rem):
      v = buf[rr, pl.ds(full * UNIT + k * L, L)]
      m = v > accs[k]
      accs[k] = jnp.maximum(accs[k], v)
      idxs[k] = jnp.where(m, vstep, idxs[k])
    for p in range(NPAIR):
      maxacc[rr, pl.ds(p * L, L)] = accs[p]
      idxacc[rr, pl.ds(p * L, L)] = idxs[p]


def _sc_argmax_body(x_hbm, xtail_hbm, pred_hbm, b0, b1, b2, b3, b4, b5,
                    tailbuf, maxacc, idxacc, pred_v,
                    s0, s1, s2, s3, s4, s5):
  wid = lax.axis_index("s") * NC + lax.axis_index("c")
  row8 = wid * G
  bufs = (b0, b1, b2, b3, b4, b5)
  sems = (s0, s1, s2, s3, s4, s5)

  def start(c, buf, sem):
    pltpu.make_async_copy(
        x_hbm.at[pl.ds(row8, G), pl.ds(c * CC, CC)], buf, sem).start()

  def wait(c, buf, sem):
    pltpu.make_async_copy(
        x_hbm.at[pl.ds(row8, G), pl.ds(c * CC, CC)], buf, sem).wait()

  neg_inf = jnp.full((L,), -jnp.inf, jnp.float32)
  zero = jnp.zeros((L,), jnp.int32)
  for rr in range(G):
    for p in range(NPAIR):
      maxacc[rr, pl.ds(p * L, L)] = neg_inf
      idxacc[rr, pl.ds(p * L, L)] = zero

  for j in range(NBUF - 1):
    start(j, bufs[j], sems[j])

  def ring_block(cb, carry):
    for j in range(NBUF):
      c = cb * NBUF + j
      wait(c, bufs[j], sems[j])
      nc = c + (NBUF - 1)
      jp = (j + NBUF - 1) % NBUF

      @pl.when(nc < NCH)
      def _(nc=nc, jp=jp):
        start(nc, bufs[jp], sems[jp])

      _scan_rows(bufs[j], maxacc, idxacc, c * CC, CC)
    return carry

  lax.fori_loop(0, NCH // NBUF, ring_block, 0)

  pltpu.make_async_copy(
      x_hbm.at[pl.ds(row8, G), pl.ds(NCH * CC, LASTC)],
      bufs[0].at[:, pl.ds(0, LASTC)], sems[0]).start()
  pltpu.make_async_copy(xtail_hbm.at[pl.ds(row8, G)], tailbuf, sems[1]).start()
  pltpu.make_async_copy(
      x_hbm.at[pl.ds(row8, G), pl.ds(NCH * CC, LASTC)],
      bufs[0].at[:, pl.ds(0, LASTC)], sems[0]).wait()
  _scan_rows(bufs[0], maxacc, idxacc, NCH * CC, LASTC)
  pltpu.make_async_copy(xtail_hbm.at[pl.ds(row8, G)], tailbuf, sems[1]).wait()
  _scan_rows(tailbuf, maxacc, idxacc, NCH * CC + LASTC, TAILW)

  lane_iota = lax.iota(jnp.int32, L)
  big = jnp.full((L,), _INT_MAX, jnp.int32)

  def merge(a, i, b, j):
    t = (b > a) | ((b == a) & (j < i))
    return jnp.where(t, b, a), jnp.where(t, j, i)

  pv = jnp.zeros((L,), jnp.int32)
  for rr in range(G):
    accs = [maxacc[rr, pl.ds(p * L, L)] for p in range(NPAIR)]
    idxs = [idxacc[rr, pl.ds(p * L, L)] * UNIT + (lane_iota + p * L)
            for p in range(NPAIR)]
    while len(accs) > 1:
      na, ni = [], []
      for q in range(0, len(accs), 2):
        a, i = merge(accs[q], idxs[q], accs[q + 1], idxs[q + 1])
        na.append(a)
        ni.append(i)
      accs, idxs = na, ni
    m = lax.reduce_max(accs[0], (0,))
    cand = jnp.where(accs[0] == m, idxs[0], big)
    idx = lax.reduce_min(cand, (0,))
    pv = jnp.where(lane_iota == rr, idx, pv)

  pred_v[pl.ds(0, L)] = pv
  pltpu.sync_copy(pred_v.at[pl.ds(0, G)], pred_hbm.at[pl.ds(row8, G)])


def _sc_gather_body(pred_hbm, lab_hbm, out_hbm, pred_v, lab_v, sem):
  wid = lax.axis_index("s") * NC + lax.axis_index("c")
  bpw = BATCH // NW
  base = wid * bpw
  pltpu.sync_copy(pred_hbm.at[pl.ds(base, bpw)], pred_v)
  pltpu.async_copy(lab_hbm.at[pred_v], lab_v, sem).wait()
  pltpu.sync_copy(lab_v, out_hbm.at[pl.ds(base, bpw)])


TC_CHUNKS = [12544] * 7 + [12160]


def _tc_argmax_kernel(x_any, xtail_any, out_ref, b0, b1, b2, b3, tb,
                      s0, s1, s2, s3, st):
  i = pl.program_id(0)
  row8 = (i + SC_ROWS // TC_BR) * TC_BR
  bufs = (b0, b1, b2, b3)
  sems = (s0, s1, s2, s3)
  nch = len(TC_CHUNKS)
  offs = [sum(TC_CHUNKS[:c]) for c in range(nch)]

  def copy(c):
    w = TC_CHUNKS[c]
    dst = bufs[c % 4]
    if w != TC_CHUNKS[0]:
      dst = dst.at[:, pl.ds(0, w)]
    return pltpu.make_async_copy(
        x_any.at[pl.ds(row8, TC_BR), pl.ds(offs[c], w)], dst, sems[c % 4])

  for c in range(3):
    copy(c).start()
  tail_cp = pltpu.make_async_copy(xtail_any.at[pl.ds(row8, TC_BR)], tb, st)
  tail_cp.start()

  lane2d = lax.broadcasted_iota(jnp.int32, (TC_BR, 128), 1)
  vm = [jnp.full((TC_BR, 128), -jnp.inf, jnp.float32) for _ in range(TC_NACC)]
  vi = [jnp.zeros((TC_BR, 128), jnp.int32) for _ in range(TC_NACC)]

  def upd(v, s):
    a = s % TC_NACC
    m = v > vm[a]
    vm[a] = jnp.maximum(vm[a], v)
    vi[a] = jnp.where(m, jnp.full((TC_BR, 128), s, jnp.int32), vi[a])

  for c in range(nch):
    copy(c).wait()
    if c + 3 < nch:
      copy(c + 3).start()
    base = offs[c] // 128
    for k in range(TC_CHUNKS[c] // 128):
      upd(bufs[c % 4][:, k * 128:(k + 1) * 128], base + k)

  tail_cp.wait()
  vtail = jnp.concatenate(
      [tb[...], jnp.full((TC_BR, 96), -jnp.inf, jnp.float32)], axis=1)
  upd(vtail, NUM_CLASSES // 128)

  cols = [vi[a] * 128 + lane2d for a in range(TC_NACC)]

  def merge(a, i, b, jj):
    t = (b > a) | ((b == a) & (jj < i))
    return jnp.where(t, b, a), jnp.where(t, jj, i)

  while len(vm) > 1:
    nm, nc = [], []
    for q in range(0, len(vm), 2):
      x, c = merge(vm[q], cols[q], vm[q + 1], cols[q + 1])
      nm.append(x)
      nc.append(c)
    vm, cols = nm, nc
  m8 = jnp.max(vm[0], axis=1, keepdims=True)
  cand = jnp.where(vm[0] == m8, cols[0], _INT_MAX)
  i8 = jnp.min(cand, axis=1, keepdims=True)
  out_ref[...] = jnp.broadcast_to(i8, (TC_BR, 128)).reshape(1, TC_BR, 128)


@jax.jit
def _run(inputs, inputs_tail, labels_i32):
  mesh = plsc.VectorSubcoreMesh(core_axis_name="c", subcore_axis_name="s")

  sc_argmax = functools.partial(
      pl.kernel,
      out_type=jax.ShapeDtypeStruct((SC_ROWS,), jnp.int32),
      mesh=mesh,
      compiler_params=pltpu.CompilerParams(needs_layout_passes=False),
      scratch_types=(
          [pltpu.VMEM((G, CC), jnp.float32)] * NBUF + [
              pltpu.VMEM((G, TAILW), jnp.float32),
              pltpu.VMEM((G, NPAIR * L), jnp.float32),
              pltpu.VMEM((G, NPAIR * L), jnp.int32),
              pltpu.VMEM((L,), jnp.int32),
          ] + [pltpu.SemaphoreType.DMA] * NBUF
      ),
  )(_sc_argmax_body)
  pred_sc = sc_argmax(inputs, inputs_tail) if SC_ROWS else None

  pred_tc = pl.pallas_call(
      _tc_argmax_kernel,
      grid=(TC_NI,),
      in_specs=[pl.BlockSpec(memory_space=pl.ANY),
                pl.BlockSpec(memory_space=pl.ANY)],
      out_specs=pl.BlockSpec((1, TC_BR, 128), lambda i: (i, 0, 0)),
      out_shape=jax.ShapeDtypeStruct((TC_NI, TC_BR, 128), jnp.int32),
      scratch_shapes=(
          [pltpu.VMEM((TC_BR, TC_CHUNKS[0]), jnp.float32)] * 4 + [
              pltpu.VMEM((TC_BR, TAILW), jnp.float32),
          ] + [pltpu.SemaphoreType.DMA] * 5
      ),
      compiler_params=pltpu.CompilerParams(
          dimension_semantics=("arbitrary",)),
  )(inputs, inputs_tail)

  pred_tc_flat = pred_tc[:, :, 0].reshape(TC_ROWS)
  if SC_ROWS:
    preds = jnp.concatenate([pred_sc, pred_tc_flat])
  else:
    preds = pred_tc_flat

  if _PROBE_XLA_GATHER:
    return jnp.take(labels_i32, preds, axis=0)
  sc_gather = functools.partial(
      pl.kernel,
      out_type=jax.ShapeDtypeStruct((BATCH,), jnp.int32),
      mesh=mesh,
      compiler_params=pltpu.CompilerParams(needs_layout_passes=False),
      scratch_types=[
          pltpu.VMEM((BATCH // NW,), jnp.int32),
          pltpu.VMEM((BATCH // NW,), jnp.int32),
          pltpu.SemaphoreType.DMA,
      ],
  )(_sc_gather_body)
  return sc_gather(preds, labels_i32)


def kernel(inputs, labels):
  inputs_tail = inputs[:, NUM_CLASSES - TAILW:]
  out = _run(inputs, inputs_tail, labels.astype(jnp.int32))
  return out.astype(labels.dtype)


# speedup vs baseline: 1.7157x; 1.7157x over previous
import functools

import jax
import jax.numpy as jnp
from jax import lax
from jax.experimental import pallas as pl
from jax.experimental.pallas import tpu as pltpu
from jax.experimental.pallas import tpu_sc as plsc

BATCH = 1024
NUM_CLASSES = 100000

SC_ROWS = 0
_PROBE_XLA_GATHER = True
TC_ROWS = BATCH - SC_ROWS

NC = 2
NS = 16
L = 16
NW = NC * NS
G = 8
NBUF = 6
CC = 1280
NCH = 78
LASTC = 128
TAILW = 32
NPAIR = 8
UNIT = 8 * L
_INT_MAX = 0x7FFFFFFF

TC_BR = 8
TC_BC = 16384
TC_NJ = -(-NUM_CLASSES // TC_BC)
TC_NI = TC_ROWS // TC_BR
TC_NACC = 8
TC_BAND = 32


def _scan_rows(buf, maxacc, idxacc, colbase, cols):
  full = cols // UNIT
  rem = (cols - full * UNIT) // L
  step0 = colbase // UNIT
  for rr in range(G):
    accs = tuple(maxacc[rr, pl.ds(p * L, L)] for p in range(NPAIR))
    idxs = tuple(idxacc[rr, pl.ds(p * L, L)] for p in range(NPAIR))
    vstep = jnp.full((L,), step0, jnp.int32)

    def body(off, st, rr=rr):
      accs, idxs, vstep = st
      accs, idxs = list(accs), list(idxs)
      for k in range(NPAIR):
        v = buf[rr, pl.ds(off + k * L, L)]
        m = v > accs[k]
        accs[k] = jnp.maximum(accs[k], v)
        idxs[k] = jnp.where(m, vstep, idxs[k])
      return (tuple(accs), tuple(idxs), vstep + 1)

    if full > 0:
      accs, idxs, vstep = plsc.parallel_loop(
          0, full * UNIT, step=UNIT, carry=(accs, idxs, vstep))(body)
    accs, idxs = list(accs), list(idxs)
    for k in range(rem):
      v = buf[rr, pl.ds(full * UNIT + k * L, L)]
      m = v > accs[k]
      accs[k] = jnp.maximum(accs[k], v)
      idxs[k] = jnp.where(m, vstep, idxs[k])
    for p in range(NPAIR):
      maxacc[rr, pl.ds(p * L, L)] = accs[p]
      idxacc[rr, pl.ds(p * L, L)] = idxs[p]


def _sc_argmax_body(x_hbm, xtail_hbm, pred_hbm, b0, b1, b2, b3, b4, b5,
                    tailbuf, maxacc, idxacc, pred_v,
                    s0, s1, s2, s3, s4, s5):
  wid = lax.axis_index("s") * NC + lax.axis_index("c")
  row8 = wid * G
  bufs = (b0, b1, b2, b3, b4, b5)
  sems = (s0, s1, s2, s3, s4, s5)

  def start(c, buf, sem):
    pltpu.make_async_copy(
        x_hbm.at[pl.ds(row8, G), pl.ds(c * CC, CC)], buf, sem).start()

  def wait(c, buf, sem):
    pltpu.make_async_copy(
        x_hbm.at[pl.ds(row8, G), pl.ds(c * CC, CC)], buf, sem).wait()

  neg_inf = jnp.full((L,), -jnp.inf, jnp.float32)
  zero = jnp.zeros((L,), jnp.int32)
  for rr in range(G):
    for p in range(NPAIR):
      maxacc[rr, pl.ds(p * L, L)] = neg_inf
      idxacc[rr, pl.ds(p * L, L)] = zero

  for j in range(NBUF - 1):
    start(j, bufs[j], sems[j])

  def ring_block(cb, carry):
    for j in range(NBUF):
      c = cb * NBUF + j
      wait(c, bufs[j], sems[j])
      nc = c + (NBUF - 1)
      jp = (j + NBUF - 1) % NBUF

      @pl.when(nc < NCH)
      def _(nc=nc, jp=jp):
        start(nc, bufs[jp], sems[jp])

      _scan_rows(bufs[j], maxacc, idxacc, c * CC, CC)
    return carry

  lax.fori_loop(0, NCH // NBUF, ring_block, 0)

  pltpu.make_async_copy(
      x_hbm.at[pl.ds(row8, G), pl.ds(NCH * CC, LASTC)],
      bufs[0].at[:, pl.ds(0, LASTC)], sems[0]).start()
  pltpu.make_async_copy(xtail_hbm.at[pl.ds(row8, G)], tailbuf, sems[1]).start()
  pltpu.make_async_copy(
      x_hbm.at[pl.ds(row8, G), pl.ds(NCH * CC, LASTC)],
      bufs[0].at[:, pl.ds(0, LASTC)], sems[0]).wait()
  _scan_rows(bufs[0], maxacc, idxacc, NCH * CC, LASTC)
  pltpu.make_async_copy(xtail_hbm.at[pl.ds(row8, G)], tailbuf, sems[1]).wait()
  _scan_rows(tailbuf, maxacc, idxacc, NCH * CC + LASTC, TAILW)

  lane_iota = lax.iota(jnp.int32, L)
  big = jnp.full((L,), _INT_MAX, jnp.int32)

  def merge(a, i, b, j):
    t = (b > a) | ((b == a) & (j < i))
    return jnp.where(t, b, a), jnp.where(t, j, i)

  pv = jnp.zeros((L,), jnp.int32)
  for rr in range(G):
    accs = [maxacc[rr, pl.ds(p * L, L)] for p in range(NPAIR)]
    idxs = [idxacc[rr, pl.ds(p * L, L)] * UNIT + (lane_iota + p * L)
            for p in range(NPAIR)]
    while len(accs) > 1:
      na, ni = [], []
      for q in range(0, len(accs), 2):
        a, i = merge(accs[q], idxs[q], accs[q + 1], idxs[q + 1])
        na.append(a)
        ni.append(i)
      accs, idxs = na, ni
    m = lax.reduce_max(accs[0], (0,))
    cand = jnp.where(accs[0] == m, idxs[0], big)
    idx = lax.reduce_min(cand, (0,))
    pv = jnp.where(lane_iota == rr, idx, pv)

  pred_v[pl.ds(0, L)] = pv
  pltpu.sync_copy(pred_v.at[pl.ds(0, G)], pred_hbm.at[pl.ds(row8, G)])


def _sc_gather_body(pred_hbm, lab_hbm, out_hbm, pred_v, lab_v, sem):
  wid = lax.axis_index("s") * NC + lax.axis_index("c")
  bpw = BATCH // NW
  base = wid * bpw
  pltpu.sync_copy(pred_hbm.at[pl.ds(base, bpw)], pred_v)
  pltpu.async_copy(lab_hbm.at[pred_v], lab_v, sem).wait()
  pltpu.sync_copy(lab_v, out_hbm.at[pl.ds(base, bpw)])


def _tc_argmax_kernel(x_ref, xtail_ref, out_ref):
  lane2d = lax.broadcasted_iota(jnp.int32, (TC_BR, 128), 1)
  nfull = NUM_CLASSES // 128

  def merge(a, i, b, jj):
    t = (b > a) | ((b == a) & (jj < i))
    return jnp.where(t, b, a), jnp.where(t, jj, i)

  pieces = []
  for bb in range(TC_BAND // TC_BR):
    r0 = bb * TC_BR
    vm = [jnp.full((TC_BR, 128), -jnp.inf, jnp.float32)
          for _ in range(TC_NACC)]
    vi = [jnp.zeros((TC_BR, 128), jnp.int32) for _ in range(TC_NACC)]

    def upd(v, s):
      a = s % TC_NACC
      m = v > vm[a]
      vm[a] = jnp.maximum(vm[a], v)
      vi[a] = jnp.where(m, jnp.full((TC_BR, 128), s, jnp.int32), vi[a])

    for s in range(nfull):
      upd(x_ref[r0:r0 + TC_BR, s * 128:(s + 1) * 128], s)
    vtail = jnp.concatenate(
        [xtail_ref[r0:r0 + TC_BR, :],
         jnp.full((TC_BR, 96), -jnp.inf, jnp.float32)], axis=1)
    upd(vtail, nfull)

    cols = [vi[a] * 128 + lane2d for a in range(TC_NACC)]
    while len(vm) > 1:
      nm, nc = [], []
      for q in range(0, len(vm), 2):
        x, c = merge(vm[q], cols[q], vm[q + 1], cols[q + 1])
        nm.append(x)
        nc.append(c)
      vm, cols = nm, nc
    m8 = jnp.max(vm[0], axis=1, keepdims=True)
    cand = jnp.where(vm[0] == m8, cols[0], _INT_MAX)
    i8 = jnp.min(cand, axis=1, keepdims=True)
    pieces.append(jnp.broadcast_to(i8, (TC_BR, 128)))
  out_ref[...] = jnp.concatenate(pieces, axis=0).reshape(1, TC_BAND, 128)


@jax.jit
def _run(inputs, inputs_tail, labels_i32):
  mesh = plsc.VectorSubcoreMesh(core_axis_name="c", subcore_axis_name="s")

  sc_argmax = functools.partial(
      pl.kernel,
      out_type=jax.ShapeDtypeStruct((SC_ROWS,), jnp.int32),
      mesh=mesh,
      compiler_params=pltpu.CompilerParams(needs_layout_passes=False),
      scratch_types=(
          [pltpu.VMEM((G, CC), jnp.float32)] * NBUF + [
              pltpu.VMEM((G, TAILW), jnp.float32),
              pltpu.VMEM((G, NPAIR * L), jnp.float32),
              pltpu.VMEM((G, NPAIR * L), jnp.int32),
              pltpu.VMEM((L,), jnp.int32),
          ] + [pltpu.SemaphoreType.DMA] * NBUF
      ),
  )(_sc_argmax_body)
  pred_sc = sc_argmax(inputs, inputs_tail) if SC_ROWS else None

  nb = TC_ROWS // TC_BAND
  pred_tc = pl.pallas_call(
      _tc_argmax_kernel,
      grid=(nb,),
      in_specs=[pl.BlockSpec((TC_BAND, NUM_CLASSES),
                             lambda i: (i + SC_ROWS // TC_BAND, 0)),
                pl.BlockSpec((TC_BAND, TAILW),
                             lambda i: (i + SC_ROWS // TC_BAND, 0))],
      out_specs=pl.BlockSpec((1, TC_BAND, 128), lambda i: (i, 0, 0)),
      out_shape=jax.ShapeDtypeStruct((nb, TC_BAND, 128), jnp.int32),
      compiler_params=pltpu.CompilerParams(
          dimension_semantics=("arbitrary",)),
  )(inputs, inputs_tail)

  pred_tc_flat = pred_tc[:, :, 0].reshape(TC_ROWS)
  if SC_ROWS:
    preds = jnp.concatenate([pred_sc, pred_tc_flat])
  else:
    preds = pred_tc_flat

  if _PROBE_XLA_GATHER:
    return jnp.take(labels_i32, preds, axis=0)
  sc_gather = functools.partial(
      pl.kernel,
      out_type=jax.ShapeDtypeStruct((BATCH,), jnp.int32),
      mesh=mesh,
      compiler_params=pltpu.CompilerParams(needs_layout_passes=False),
      scratch_types=[
          pltpu.VMEM((BATCH // NW,), jnp.int32),
          pltpu.VMEM((BATCH // NW,), jnp.int32),
          pltpu.SemaphoreType.DMA,
      ],
  )(_sc_gather_body)
  return sc_gather(preds, labels_i32)


def kernel(inputs, labels):
  inputs_tail = inputs[:, NUM_CLASSES - TAILW:]
  out = _run(inputs, inputs_tail, labels.astype(jnp.int32))
  return out.astype(labels.dtype)
